# baseline (speedup 1.0000x reference)
import jax
import jax.numpy as jnp
from jax import lax
from jax.experimental import pallas as pl

B, C, H, W_SP, E, K = 4, 64, 128, 128, 64, 8
S = H * W_SP
T = B * S
HB = 16
TBLK = HB * W_SP
NH = H // HB


def _router_body(x_ref, w_ref, b_ref, logits_ref, weights_ref, idx_ref,
                 mask_ref):
    xb = x_ref[0].reshape(C, TBLK)
    l = jnp.dot(w_ref[...], xb, preferred_element_type=jnp.float32)
    l = l + b_ref[...]
    logits_ref[0] = l

    m = jnp.max(l, axis=0, keepdims=True)
    e = jnp.exp(l - m)
    z = jnp.sum(e, axis=0, keepdims=True)
    p = e / z

    ii = lax.broadcasted_iota(jnp.int32, (C, TBLK), 0)
    vals, idxs = [], []
    cur = p
    for _ in range(K):
        mk = jnp.max(cur, axis=0, keepdims=True)
        sel = cur == mk
        ik = jnp.min(jnp.where(sel, ii, C), axis=0, keepdims=True)
        vals.append(mk)
        idxs.append(ik)
        cur = jnp.where(ii == ik, -1.0, cur)

    wv = jnp.concatenate(vals, axis=0)
    iv = jnp.concatenate(idxs, axis=0)
    weights_ref[0] = wv / jnp.sum(wv, axis=0, keepdims=True)
    idx_ref[0] = iv

    ee = lax.broadcasted_iota(jnp.int32, (E, K, TBLK), 0)
    mask_ref[...] = (iv[None] == ee).astype(jnp.int32)


def kernel(x, W, b):
    br = b.reshape(C, 1)
    logits, weights, idx, mask = pl.pallas_call(
        _router_body,
        grid=(B, NH),
        in_specs=[
            pl.BlockSpec((1, C, HB, W_SP), lambda bb, h: (bb, 0, h, 0)),
            pl.BlockSpec((C, C), lambda bb, h: (0, 0)),
            pl.BlockSpec((C, 1), lambda bb, h: (0, 0)),
        ],
        out_specs=[
            pl.BlockSpec((1, C, TBLK), lambda bb, h: (bb, 0, h)),
            pl.BlockSpec((1, K, TBLK), lambda bb, h: (bb, 0, h)),
            pl.BlockSpec((1, K, TBLK), lambda bb, h: (bb, 0, h)),
            pl.BlockSpec((E, K, TBLK), lambda bb, h: (0, 0, bb * NH + h)),
        ],
        out_shape=[
            jax.ShapeDtypeStruct((B, C, S), jnp.float32),
            jax.ShapeDtypeStruct((B, K, S), jnp.float32),
            jax.ShapeDtypeStruct((B, K, S), jnp.int32),
            jax.ShapeDtypeStruct((E, K, T), jnp.int32),
        ],
    )(x, W, br)
    return (
        logits.reshape(B, C, H, W_SP),
        weights.reshape(B, K, H, W_SP),
        idx.reshape(B, K, H, W_SP),
        mask,
    )


# speedup vs baseline: 1.3948x; 1.3948x over previous
import jax
import jax.numpy as jnp
from jax import lax
from jax.experimental import pallas as pl

B, C, H, W_SP, E, K = 4, 64, 128, 128, 64, 8
T = B * H * W_SP
HB = 16
TBLK = HB * W_SP
NH = H // HB


def _router_body(x_ref, w_ref, b_ref, logits_ref, weights_ref, idx_ref,
                 mask_ref):
    xb = x_ref[0].reshape(C, TBLK)
    l = jnp.dot(w_ref[...], xb, preferred_element_type=jnp.float32)
    l = l + b_ref[...]
    l3 = l.reshape(C, HB, W_SP)
    logits_ref[0] = l3

    m = jnp.max(l3, axis=0, keepdims=True)
    e = jnp.exp(l3 - m)
    z = jnp.sum(e, axis=0, keepdims=True)
    p = e / z

    ii = lax.broadcasted_iota(jnp.int32, (C, HB, W_SP), 0)
    vals, idxs = [], []
    cur = p
    for _ in range(K):
        mk = jnp.max(cur, axis=0, keepdims=True)
        sel = cur == mk
        ik = jnp.min(jnp.where(sel, ii, C), axis=0, keepdims=True)
        vals.append(mk)
        idxs.append(ik)
        cur = jnp.where(ii == ik, -1.0, cur)

    wv = jnp.concatenate(vals, axis=0)
    iv = jnp.concatenate(idxs, axis=0)
    weights_ref[0] = wv / jnp.sum(wv, axis=0, keepdims=True)
    idx_ref[0] = iv

    iv2 = iv.reshape(K, TBLK)
    ee = lax.broadcasted_iota(jnp.int32, (E, K, TBLK), 0)
    mask_ref[...] = (iv2[None] == ee).astype(jnp.int32)


def kernel(x, W, b):
    br = b.reshape(C, 1)
    logits, weights, idx, mask = pl.pallas_call(
        _router_body,
        grid=(B, NH),
        in_specs=[
            pl.BlockSpec((1, C, HB, W_SP), lambda bb, h: (bb, 0, h, 0)),
            pl.BlockSpec((C, C), lambda bb, h: (0, 0)),
            pl.BlockSpec((C, 1), lambda bb, h: (0, 0)),
        ],
        out_specs=[
            pl.BlockSpec((1, C, HB, W_SP), lambda bb, h: (bb, 0, h, 0)),
            pl.BlockSpec((1, K, HB, W_SP), lambda bb, h: (bb, 0, h, 0)),
            pl.BlockSpec((1, K, HB, W_SP), lambda bb, h: (bb, 0, h, 0)),
            pl.BlockSpec((E, K, TBLK), lambda bb, h: (0, 0, bb * NH + h)),
        ],
        out_shape=[
            jax.ShapeDtypeStruct((B, C, H, W_SP), jnp.float32),
            jax.ShapeDtypeStruct((B, K, H, W_SP), jnp.float32),
            jax.ShapeDtypeStruct((B, K, H, W_SP), jnp.int32),
            jax.ShapeDtypeStruct((E, K, T), jnp.int32),
        ],
    )(x, W, br)
    return (logits, weights, idx, mask)


# speedup vs baseline: 1.4913x; 1.0691x over previous
import jax
import jax.numpy as jnp
from jax import lax
from jax.experimental import pallas as pl

B, C, H, W_SP, E, K = 4, 64, 128, 128, 64, 8
T = B * H * W_SP
HB = 16
TBLK = HB * W_SP
NH = H // HB


def _router_body(x_ref, w_ref, b_ref, logits_ref, weights_ref, idx_ref,
                 mask_ref):
    xb = x_ref[0].reshape(C, TBLK)
    l = jnp.dot(w_ref[...], xb, preferred_element_type=jnp.float32)
    l = l + b_ref[...]
    l3 = l.reshape(C, HB, W_SP)
    logits_ref[0] = l3

    cur = [l3[c] for c in range(C)]
    vals, idxs = [], []
    neg = jnp.float32(-jnp.inf)
    for _ in range(K):
        bv, bi = cur[0], jnp.zeros((HB, W_SP), jnp.int32)
        for c in range(1, C):
            take = cur[c] > bv
            bv = jnp.where(take, cur[c], bv)
            bi = jnp.where(take, jnp.int32(c), bi)
        vals.append(bv)
        idxs.append(bi)
        for c in range(C):
            cur[c] = jnp.where(bi == c, neg, cur[c])

    es = [jnp.exp(v - vals[0]) for v in vals]
    z = es[0]
    for e in es[1:]:
        z = z + e
    zr = 1.0 / z
    wv = jnp.stack([e * zr for e in es], axis=0)
    iv = jnp.stack(idxs, axis=0)
    weights_ref[0] = wv
    idx_ref[0] = iv

    iv2 = iv.reshape(K, TBLK)
    ee = lax.broadcasted_iota(jnp.int32, (E, K, TBLK), 0)
    mask_ref[...] = (iv2[None] == ee).astype(jnp.int32)


def kernel(x, W, b):
    br = b.reshape(C, 1)
    logits, weights, idx, mask = pl.pallas_call(
        _router_body,
        grid=(B, NH),
        in_specs=[
            pl.BlockSpec((1, C, HB, W_SP), lambda bb, h: (bb, 0, h, 0)),
            pl.BlockSpec((C, C), lambda bb, h: (0, 0)),
            pl.BlockSpec((C, 1), lambda bb, h: (0, 0)),
        ],
        out_specs=[
            pl.BlockSpec((1, C, HB, W_SP), lambda bb, h: (bb, 0, h, 0)),
            pl.BlockSpec((1, K, HB, W_SP), lambda bb, h: (bb, 0, h, 0)),
            pl.BlockSpec((1, K, HB, W_SP), lambda bb, h: (bb, 0, h, 0)),
            pl.BlockSpec((E, K, TBLK), lambda bb, h: (0, 0, bb * NH + h)),
        ],
        out_shape=[
            jax.ShapeDtypeStruct((B, C, H, W_SP), jnp.float32),
            jax.ShapeDtypeStruct((B, K, H, W_SP), jnp.float32),
            jax.ShapeDtypeStruct((B, K, H, W_SP), jnp.int32),
            jax.ShapeDtypeStruct((E, K, T), jnp.int32),
        ],
    )(x, W, br)
    return (logits, weights, idx, mask)


# speedup vs baseline: 1.5666x; 1.0505x over previous
import jax
import jax.numpy as jnp
from jax import lax
from jax.experimental import pallas as pl
from jax.experimental.pallas import tpu as pltpu

B, C, H, W_SP, E, K = 4, 64, 128, 128, 64, 8
T = B * H * W_SP
HB = 16
TBLK = HB * W_SP
NH = H // HB


def _router_body(x_ref, w_ref, b_ref, logits_ref, weights_ref, idx_ref,
                 mask_ref):
    xb = x_ref[0].reshape(C, TBLK)
    l = jnp.dot(w_ref[...], xb, preferred_element_type=jnp.float32)
    l3 = l.reshape(C, HB, W_SP)

    cur = []
    for c in range(C):
        lc = l3[c] + b_ref[c]
        logits_ref[0, c] = lc
        cur.append(lc)

    vals, idxs = [], []
    neg = jnp.float32(-jnp.inf)
    for _ in range(K):
        bv, bi = cur[0], jnp.zeros((HB, W_SP), jnp.int32)
        for c in range(1, C):
            take = cur[c] > bv
            bv = jnp.where(take, cur[c], bv)
            bi = jnp.where(take, jnp.int32(c), bi)
        vals.append(bv)
        idxs.append(bi)
        for c in range(C):
            cur[c] = jnp.where(bi == c, neg, cur[c])

    es = [jnp.exp(v - vals[0]) for v in vals]
    z = es[0]
    for e in es[1:]:
        z = z + e
    zr = 1.0 / z
    wv = jnp.stack([e * zr for e in es], axis=0)
    iv = jnp.stack(idxs, axis=0)
    weights_ref[0] = wv
    idx_ref[0] = iv

    iv2 = iv.reshape(K, TBLK)
    ee = lax.broadcasted_iota(jnp.int32, (E, K, TBLK), 0)
    mask_ref[...] = (iv2[None] == ee).astype(jnp.int32)


def kernel(x, W, b):
    logits, weights, idx, mask = pl.pallas_call(
        _router_body,
        grid=(B, NH),
        in_specs=[
            pl.BlockSpec((1, C, HB, W_SP), lambda bb, h: (bb, 0, h, 0)),
            pl.BlockSpec((C, C), lambda bb, h: (0, 0)),
            pl.BlockSpec(memory_space=pltpu.SMEM),
        ],
        out_specs=[
            pl.BlockSpec((1, C, HB, W_SP), lambda bb, h: (bb, 0, h, 0)),
            pl.BlockSpec((1, K, HB, W_SP), lambda bb, h: (bb, 0, h, 0)),
            pl.BlockSpec((1, K, HB, W_SP), lambda bb, h: (bb, 0, h, 0)),
            pl.BlockSpec((E, K, TBLK), lambda bb, h: (0, 0, bb * NH + h)),
        ],
        out_shape=[
            jax.ShapeDtypeStruct((B, C, H, W_SP), jnp.float32),
            jax.ShapeDtypeStruct((B, K, H, W_SP), jnp.float32),
            jax.ShapeDtypeStruct((B, K, H, W_SP), jnp.int32),
            jax.ShapeDtypeStruct((E, K, T), jnp.int32),
        ],
    )(x, W, b)
    return (logits, weights, idx, mask)


# speedup vs baseline: 1.7167x; 1.0958x over previous
import jax
import jax.numpy as jnp
from jax import lax
from jax.experimental import pallas as pl
from jax.experimental.pallas import tpu as pltpu

B, C, H, W_SP, E, K = 4, 64, 128, 128, 64, 8
T = B * H * W_SP
HB = 32
TBLK = HB * W_SP
NH = H // HB


def _router_body(x_ref, w_ref, b_ref, logits_ref, weights_ref, idx_ref,
                 mask_ref):
    xb = x_ref[0].reshape(C, TBLK)
    l = jnp.dot(w_ref[...], xb, preferred_element_type=jnp.float32)
    l3 = l.reshape(C, HB, W_SP)

    cur = []
    for c in range(C):
        lc = l3[c] + b_ref[c]
        logits_ref[0, c] = lc
        cur.append(lc)

    vals, idxs = [], []
    neg = jnp.float32(-jnp.inf)
    for _ in range(K):
        bv, bi = cur[0], jnp.zeros((HB, W_SP), jnp.int32)
        for c in range(1, C):
            take = cur[c] > bv
            bv = jnp.where(take, cur[c], bv)
            bi = jnp.where(take, jnp.int32(c), bi)
        vals.append(bv)
        idxs.append(bi)
        for c in range(C):
            cur[c] = jnp.where(bi == c, neg, cur[c])

    es = [jnp.exp(v - vals[0]) for v in vals]
    z = es[0]
    for e in es[1:]:
        z = z + e
    zr = 1.0 / z
    wv = jnp.stack([e * zr for e in es], axis=0)
    iv = jnp.stack(idxs, axis=0)
    weights_ref[0] = wv
    idx_ref[0] = iv

    iv2 = iv.reshape(K, TBLK)
    ee = lax.broadcasted_iota(jnp.int32, (E, K, TBLK), 0)
    mask_ref[...] = (iv2[None] == ee).astype(jnp.int32)


def kernel(x, W, b):
    logits, weights, idx, mask = pl.pallas_call(
        _router_body,
        grid=(B, NH),
        in_specs=[
            pl.BlockSpec((1, C, HB, W_SP), lambda bb, h: (bb, 0, h, 0)),
            pl.BlockSpec((C, C), lambda bb, h: (0, 0)),
            pl.BlockSpec(memory_space=pltpu.SMEM),
        ],
        out_specs=[
            pl.BlockSpec((1, C, HB, W_SP), lambda bb, h: (bb, 0, h, 0)),
            pl.BlockSpec((1, K, HB, W_SP), lambda bb, h: (bb, 0, h, 0)),
            pl.BlockSpec((1, K, HB, W_SP), lambda bb, h: (bb, 0, h, 0)),
            pl.BlockSpec((E, K, TBLK), lambda bb, h: (0, 0, bb * NH + h)),
        ],
        out_shape=[
            jax.ShapeDtypeStruct((B, C, H, W_SP), jnp.float32),
            jax.ShapeDtypeStruct((B, K, H, W_SP), jnp.float32),
            jax.ShapeDtypeStruct((B, K, H, W_SP), jnp.int32),
            jax.ShapeDtypeStruct((E, K, T), jnp.int32),
        ],
    )(x, W, b)
    return (logits, weights, idx, mask)


# speedup vs baseline: 1.7489x; 1.0188x over previous
import jax
import jax.numpy as jnp
from jax import lax
from jax.experimental import pallas as pl
from jax.experimental.pallas import tpu as pltpu

B, C, H, W_SP, E, K = 4, 64, 128, 128, 64, 8
T = B * H * W_SP
HB = 64
TBLK = HB * W_SP
NH = H // HB


def _router_body(x_ref, w_ref, b_ref, logits_ref, weights_ref, idx_ref,
                 mask_ref):
    xb = x_ref[0].reshape(C, TBLK)
    l = jnp.dot(w_ref[...], xb, preferred_element_type=jnp.float32)
    l3 = l.reshape(C, HB, W_SP)

    cur = []
    for c in range(C):
        lc = l3[c] + b_ref[c]
        logits_ref[0, c] = lc
        cur.append(lc)

    vals, idxs = [], []
    neg = jnp.float32(-jnp.inf)
    for _ in range(K):
        bv, bi = cur[0], jnp.zeros((HB, W_SP), jnp.int32)
        for c in range(1, C):
            take = cur[c] > bv
            bv = jnp.where(take, cur[c], bv)
            bi = jnp.where(take, jnp.int32(c), bi)
        vals.append(bv)
        idxs.append(bi)
        for c in range(C):
            cur[c] = jnp.where(bi == c, neg, cur[c])

    es = [jnp.exp(v - vals[0]) for v in vals]
    z = es[0]
    for e in es[1:]:
        z = z + e
    zr = 1.0 / z
    wv = jnp.stack([e * zr for e in es], axis=0)
    iv = jnp.stack(idxs, axis=0)
    weights_ref[0] = wv
    idx_ref[0] = iv

    iv2 = iv.reshape(K, TBLK)
    ee = lax.broadcasted_iota(jnp.int32, (E, K, TBLK), 0)
    mask_ref[...] = (iv2[None] == ee).astype(jnp.int32)


def kernel(x, W, b):
    logits, weights, idx, mask = pl.pallas_call(
        _router_body,
        grid=(B, NH),
        in_specs=[
            pl.BlockSpec((1, C, HB, W_SP), lambda bb, h: (bb, 0, h, 0)),
            pl.BlockSpec((C, C), lambda bb, h: (0, 0)),
            pl.BlockSpec(memory_space=pltpu.SMEM),
        ],
        out_specs=[
            pl.BlockSpec((1, C, HB, W_SP), lambda bb, h: (bb, 0, h, 0)),
            pl.BlockSpec((1, K, HB, W_SP), lambda bb, h: (bb, 0, h, 0)),
            pl.BlockSpec((1, K, HB, W_SP), lambda bb, h: (bb, 0, h, 0)),
            pl.BlockSpec((E, K, TBLK), lambda bb, h: (0, 0, bb * NH + h)),
        ],
        out_shape=[
            jax.ShapeDtypeStruct((B, C, H, W_SP), jnp.float32),
            jax.ShapeDtypeStruct((B, K, H, W_SP), jnp.float32),
            jax.ShapeDtypeStruct((B, K, H, W_SP), jnp.int32),
            jax.ShapeDtypeStruct((E, K, T), jnp.int32),
        ],
    )(x, W, b)
    return (logits, weights, idx, mask)
